# Initial kernel scaffold; baseline (speedup 1.0000x reference)
import jax
import jax.numpy as jnp
from jax.experimental import pallas as pl


def kernel(feat):
    raise NotImplementedError("write your pallas kernel here")


# speedup vs baseline: 472.7768x; 472.7768x over previous
import functools

import jax
import jax.numpy as jnp
import numpy as np
from jax import lax
from jax.experimental import pallas as pl

NUMANGLE = 100
NUMRHO = 100


def _rho_table(H, W):
    irho = float(int(np.sqrt(H * H + W * W) + 1)) / float(NUMRHO - 1)
    itheta = np.pi / NUMANGLE
    angles = jnp.arange(NUMANGLE, dtype=jnp.float32) * itheta
    tabCos = jnp.cos(angles) / irho
    tabSin = jnp.sin(angles) / irho
    xs = jnp.arange(W, dtype=jnp.float32) - (W // 2)
    ys = jnp.arange(H, dtype=jnp.float32) - (H // 2)
    r = jnp.round(xs[None, None, :] * tabCos[:, None, None]
                  + ys[None, :, None] * tabSin[:, None, None]).astype(jnp.int32)
    r = r + NUMRHO // 2
    r = jnp.clip(r, 0, NUMRHO - 1)
    return r.reshape(NUMANGLE, 1, H * W)


def _dht_body(feat_ref, r_ref, out_ref):
    rv = r_ref[0, 0, :]
    onehot = (rv[:, None] == lax.broadcasted_iota(jnp.int32, (rv.shape[0], NUMRHO), 1))
    onehot = onehot.astype(jnp.float32)
    out_ref[0] = jnp.dot(feat_ref[...], onehot,
                         preferred_element_type=jnp.float32)


@functools.partial(jax.jit, static_argnames=("interpret",))
def kernel(feat, interpret=False):
    N, C, H, W = feat.shape
    P = H * W
    NC = N * C
    feat_flat = feat.reshape(NC, P)
    r = _rho_table(H, W)

    out = pl.pallas_call(
        _dht_body,
        grid=(NUMANGLE,),
        in_specs=[
            pl.BlockSpec((NC, P), lambda a: (0, 0)),
            pl.BlockSpec((1, 1, P), lambda a: (a, 0, 0)),
        ],
        out_specs=pl.BlockSpec((1, NC, NUMRHO), lambda a: (a, 0, 0)),
        out_shape=jax.ShapeDtypeStruct((NUMANGLE, NC, NUMRHO), jnp.float32),
        interpret=interpret,
    )(feat_flat, r)

    return jnp.transpose(out, (1, 0, 2)).reshape(N, C, NUMANGLE, NUMRHO)
